# Initial kernel scaffold; baseline (speedup 1.0000x reference)
import jax
import jax.numpy as jnp
from jax.experimental import pallas as pl


def kernel(x, Wg, bg, Wv, bv, Wd, bd, Wgate):
    raise NotImplementedError("write your pallas kernel here")


# speedup vs baseline: 42.4284x; 42.4284x over previous
import jax
import jax.numpy as jnp
import numpy as np
from jax.experimental import pallas as pl
from jax.experimental.pallas import tpu as pltpu

E = 4
D = 1024
T = 256


def _body(x_ref, w_hbm, wgate_ref, bg_ref, bv_ref, bd_ref, o_ref,
          w_vmem, carry_ref, sem):
    bi = pl.program_id(1)
    c = pl.program_id(2)

    @pl.when((bi == 0) & (c == 0))
    def _load_weights():
        cp = pltpu.make_async_copy(w_hbm, w_vmem, sem)
        cp.start()
        cp.wait()

    @pl.when(c == 0)
    def _reset_carry():
        carry_ref[...] = jnp.zeros_like(carry_ref)

    xb = x_ref[0].astype(jnp.bfloat16)

    logits = jnp.dot(xb, wgate_ref[...],
                     preferred_element_type=jnp.float32)[:, :E]
    m1 = jnp.max(logits, axis=-1, keepdims=True)
    masked = jnp.where(logits >= m1, -1e30, logits)
    m2 = jnp.max(masked, axis=-1, keepdims=True)
    denom = 1.0 + jnp.exp(m2 - m1)
    wts = jnp.where(logits >= m2, jnp.exp(logits - m1), 0.0) / denom

    y = jnp.zeros((T, D), jnp.float32)
    for e in range(E):
        gvd = jnp.dot(xb, w_vmem[e], preferred_element_type=jnp.float32)
        g = gvd[:, 0:D] + bg_ref[e:e + 1, :]
        v = gvd[:, D:2 * D] + bv_ref[e:e + 1, :]
        d = gvd[:, 2 * D:3 * D] + bd_ref[e:e + 1, :]
        xs = jax.nn.sigmoid(g) * jnp.tanh(v)
        a = 0.001 + 0.998 * jax.nn.sigmoid(d)

        A, X = a, xs
        k = 1
        while k < T:
            ash = jnp.concatenate(
                [jnp.ones((k, D), jnp.float32), A[:-k]], axis=0)
            xsh = jnp.concatenate(
                [jnp.zeros((k, D), jnp.float32), X[:-k]], axis=0)
            X = X + A * xsh
            A = A * ash
            k *= 2

        hprev = carry_ref[e:e + 1, :]
        out_e = X + A * hprev
        carry_ref[e:e + 1, :] = out_e[T - 1:T, :]
        y = y + out_e * wts[:, e:e + 1]
    o_ref[0] = y


def kernel(x, Wg, bg, Wv, bv, Wd, bd, Wgate):
    B, S, d_model = x.shape
    assert d_model == D and B % 2 == 0 and S % T == 0
    n_chunks = S // T

    w_stack = jnp.concatenate(
        [jnp.swapaxes(Wg, 1, 2), jnp.swapaxes(Wv, 1, 2),
         jnp.swapaxes(Wd, 1, 2)], axis=2).astype(jnp.bfloat16)
    wgate_pad = jnp.zeros((D, 128), jnp.float32).at[:, :E].set(Wgate.T)
    wgate_pad = wgate_pad.astype(jnp.bfloat16)

    grid = (2, B // 2, n_chunks)

    out = pl.pallas_call(
        _body,
        out_shape=jax.ShapeDtypeStruct((B, S, D), jnp.float32),
        grid=grid,
        in_specs=[
            pl.BlockSpec((1, T, D), lambda bo, bi, c: (bo * 2 + bi, c, 0)),
            pl.BlockSpec(memory_space=pl.ANY),
            pl.BlockSpec((D, 128), lambda bo, bi, c: (0, 0)),
            pl.BlockSpec((E, D), lambda bo, bi, c: (0, 0)),
            pl.BlockSpec((E, D), lambda bo, bi, c: (0, 0)),
            pl.BlockSpec((E, D), lambda bo, bi, c: (0, 0)),
        ],
        out_specs=pl.BlockSpec((1, T, D), lambda bo, bi, c: (bo * 2 + bi, c, 0)),
        scratch_shapes=[
            pltpu.VMEM((E, D, 3 * D), jnp.bfloat16),
            pltpu.VMEM((E, D), jnp.float32),
            pltpu.SemaphoreType.DMA,
        ],
        compiler_params=pltpu.CompilerParams(
            dimension_semantics=("parallel", "arbitrary", "arbitrary"),
            vmem_limit_bytes=56 * 1024 * 1024,
        ),
        name="moe_mingru",
    )(x, w_stack, wgate_pad, bg, bv, bd)
    return out


# speedup vs baseline: 43.2384x; 1.0191x over previous
import jax
import jax.numpy as jnp
import numpy as np
from jax.experimental import pallas as pl
from jax.experimental.pallas import tpu as pltpu

E = 4
D = 1024
T = 256


def _body(x_ref, w_hbm, wgate_ref, bg_ref, bv_ref, bd_ref, o_ref,
          w_vmem, carry_ref, sem):
    bi = pl.program_id(1)
    c = pl.program_id(2)

    @pl.when((bi == 0) & (c == 0))
    def _load_weights():
        cp = pltpu.make_async_copy(w_hbm, w_vmem, sem)
        cp.start()
        cp.wait()

    @pl.when(c == 0)
    def _reset_carry():
        carry_ref[...] = jnp.zeros_like(carry_ref)

    xb = x_ref[0].astype(jnp.bfloat16)

    logits = jnp.dot(xb, wgate_ref[...],
                     preferred_element_type=jnp.float32)[:, :E]
    m1 = jnp.max(logits, axis=-1, keepdims=True)
    masked = jnp.where(logits >= m1, -1e30, logits)
    m2 = jnp.max(masked, axis=-1, keepdims=True)
    denom = 1.0 + jnp.exp(m2 - m1)
    wts = jnp.where(logits >= m2, jnp.exp(logits - m1), 0.0) / denom

    y = jnp.zeros((T, D), jnp.float32)
    for e in range(E):
        gvd = jnp.dot(xb, w_vmem[e], preferred_element_type=jnp.float32)
        g = gvd[:, 0:D] + bg_ref[e:e + 1, :]
        v = gvd[:, D:2 * D] + bv_ref[e:e + 1, :]
        d = gvd[:, 2 * D:3 * D] + bd_ref[e:e + 1, :]
        xs = (0.5 * jnp.tanh(0.5 * g) + 0.5) * jnp.tanh(v)
        a = 0.5 + 0.499 * jnp.tanh(0.5 * d)

        A, X = a, xs
        k = 1
        while k < T:
            ash = jnp.concatenate(
                [jnp.ones((k, D), jnp.float32), A[:-k]], axis=0)
            xsh = jnp.concatenate(
                [jnp.zeros((k, D), jnp.float32), X[:-k]], axis=0)
            X = X + A * xsh
            A = A * ash
            k *= 2

        hprev = carry_ref[e:e + 1, :]
        out_e = X + A * hprev
        carry_ref[e:e + 1, :] = out_e[T - 1:T, :]
        y = y + out_e * wts[:, e:e + 1]
    o_ref[0] = y


def kernel(x, Wg, bg, Wv, bv, Wd, bd, Wgate):
    B, S, d_model = x.shape
    assert d_model == D and B % 2 == 0 and S % T == 0
    n_chunks = S // T

    w_stack = jnp.concatenate(
        [jnp.swapaxes(Wg, 1, 2), jnp.swapaxes(Wv, 1, 2),
         jnp.swapaxes(Wd, 1, 2)], axis=2).astype(jnp.bfloat16)
    wgate_pad = jnp.zeros((D, 128), jnp.float32).at[:, :E].set(Wgate.T)
    wgate_pad = wgate_pad.astype(jnp.bfloat16)

    grid = (2, B // 2, n_chunks)

    out = pl.pallas_call(
        _body,
        out_shape=jax.ShapeDtypeStruct((B, S, D), jnp.float32),
        grid=grid,
        in_specs=[
            pl.BlockSpec((1, T, D), lambda bo, bi, c: (bo * 2 + bi, c, 0)),
            pl.BlockSpec(memory_space=pl.ANY),
            pl.BlockSpec((D, 128), lambda bo, bi, c: (0, 0)),
            pl.BlockSpec((E, D), lambda bo, bi, c: (0, 0)),
            pl.BlockSpec((E, D), lambda bo, bi, c: (0, 0)),
            pl.BlockSpec((E, D), lambda bo, bi, c: (0, 0)),
        ],
        out_specs=pl.BlockSpec((1, T, D), lambda bo, bi, c: (bo * 2 + bi, c, 0)),
        scratch_shapes=[
            pltpu.VMEM((E, D, 3 * D), jnp.bfloat16),
            pltpu.VMEM((E, D), jnp.float32),
            pltpu.SemaphoreType.DMA,
        ],
        compiler_params=pltpu.CompilerParams(
            dimension_semantics=("parallel", "arbitrary", "arbitrary"),
            vmem_limit_bytes=56 * 1024 * 1024,
        ),
        name="moe_mingru",
    )(x, w_stack, wgate_pad, bg, bv, bd)
    return out


# speedup vs baseline: 50.0371x; 1.1572x over previous
import jax
import jax.numpy as jnp
import numpy as np
from jax.experimental import pallas as pl
from jax.experimental.pallas import tpu as pltpu

E = 4
D = 1024
T = 256
R = 8
G = T // R


def _body(x_ref, w_hbm, wgate_ref, bg_ref, bv_ref, bd_ref, o_ref,
          w_vmem, carry_ref, sem):
    bi = pl.program_id(1)
    c = pl.program_id(2)

    @pl.when((bi == 0) & (c == 0))
    def _load_weights():
        cp = pltpu.make_async_copy(w_hbm, w_vmem, sem)
        cp.start()
        cp.wait()

    @pl.when(c == 0)
    def _reset_carry():
        carry_ref[...] = jnp.zeros_like(carry_ref)

    xb = x_ref[0].astype(jnp.bfloat16)

    logits = jnp.dot(xb, wgate_ref[...],
                     preferred_element_type=jnp.float32)[:, :E]
    m1 = jnp.max(logits, axis=-1, keepdims=True)
    masked = jnp.where(logits >= m1, -1e30, logits)
    m2 = jnp.max(masked, axis=-1, keepdims=True)
    denom = 1.0 + jnp.exp(m2 - m1)
    wts = jnp.where(logits >= m2, jnp.exp(logits - m1), 0.0) / denom

    y = jnp.zeros((T, D), jnp.float32)
    for e in range(E):
        gvd = jnp.dot(xb, w_vmem[e], preferred_element_type=jnp.float32)
        g = gvd[:, 0:D] + bg_ref[e:e + 1, :]
        v = gvd[:, D:2 * D] + bv_ref[e:e + 1, :]
        d = gvd[:, 2 * D:3 * D] + bd_ref[e:e + 1, :]
        X = (0.5 * jnp.tanh(0.5 * g) + 0.5) * jnp.tanh(v)
        A = 0.5 + 0.499 * jnp.tanh(0.5 * d)

        kk = R
        while kk < T:
            X = jnp.concatenate([X[:kk], X[kk:] + A[kk:] * X[:T - kk]], axis=0)
            A = jnp.concatenate([A[:kk], A[kk:] * A[:T - kk]], axis=0)
            kk *= 2

        sA = A[T - R:]
        sX = X[T - R:]
        for k in (1, 2, 4):
            shA = jnp.concatenate(
                [jnp.ones((k, D), jnp.float32), sA[:R - k]], axis=0)
            shX = jnp.concatenate(
                [jnp.zeros((k, D), jnp.float32), sX[:R - k]], axis=0)
            sX = sX + sA * shX
            sA = sA * shA
        pA = jnp.concatenate([jnp.ones((1, D), jnp.float32), sA[:R - 1]], axis=0)
        pX = jnp.concatenate([jnp.zeros((1, D), jnp.float32), sX[:R - 1]], axis=0)
        hprev = carry_ref[e:e + 1, :]
        P = pX + pA * hprev

        out_e = (X.reshape(G, R, D) + A.reshape(G, R, D) * P[None, :, :]
                 ).reshape(T, D)
        carry_ref[e:e + 1, :] = out_e[T - 1:T, :]
        y = y + out_e * wts[:, e:e + 1]
    o_ref[0] = y


def kernel(x, Wg, bg, Wv, bv, Wd, bd, Wgate):
    B, S, d_model = x.shape
    assert d_model == D and B % 2 == 0 and S % T == 0
    n_chunks = S // T

    xp = (x.reshape(B, n_chunks, R, G, D)
            .swapaxes(2, 3)
            .reshape(B, S, D))

    w_stack = jnp.concatenate(
        [jnp.swapaxes(Wg, 1, 2), jnp.swapaxes(Wv, 1, 2),
         jnp.swapaxes(Wd, 1, 2)], axis=2).astype(jnp.bfloat16)
    wgate_pad = jnp.zeros((D, 128), jnp.float32).at[:, :E].set(Wgate.T)
    wgate_pad = wgate_pad.astype(jnp.bfloat16)

    grid = (2, B // 2, n_chunks)

    out = pl.pallas_call(
        _body,
        out_shape=jax.ShapeDtypeStruct((B, S, D), jnp.float32),
        grid=grid,
        in_specs=[
            pl.BlockSpec((1, T, D), lambda bo, bi, c: (bo * 2 + bi, c, 0)),
            pl.BlockSpec(memory_space=pl.ANY),
            pl.BlockSpec((D, 128), lambda bo, bi, c: (0, 0)),
            pl.BlockSpec((E, D), lambda bo, bi, c: (0, 0)),
            pl.BlockSpec((E, D), lambda bo, bi, c: (0, 0)),
            pl.BlockSpec((E, D), lambda bo, bi, c: (0, 0)),
        ],
        out_specs=pl.BlockSpec((1, T, D), lambda bo, bi, c: (bo * 2 + bi, c, 0)),
        scratch_shapes=[
            pltpu.VMEM((E, D, 3 * D), jnp.bfloat16),
            pltpu.VMEM((E, D), jnp.float32),
            pltpu.SemaphoreType.DMA,
        ],
        compiler_params=pltpu.CompilerParams(
            dimension_semantics=("parallel", "arbitrary", "arbitrary"),
            vmem_limit_bytes=56 * 1024 * 1024,
        ),
        name="moe_mingru",
    )(xp, w_stack, wgate_pad, bg, bv, bd)

    return (out.reshape(B, n_chunks, G, R, D)
               .swapaxes(2, 3)
               .reshape(B, S, D))


# speedup vs baseline: 55.5072x; 1.1093x over previous
import jax
import jax.numpy as jnp
import numpy as np
from jax.experimental import pallas as pl
from jax.experimental.pallas import tpu as pltpu

E = 4
D = 1024
T = 256
R = 8
G = T // R


def _body(x_ref, w_hbm, wgate_ref, bg_ref, bv_ref, bd_ref, o_ref,
          w_vmem, carry_ref, sem):
    bi = pl.program_id(1)
    c = pl.program_id(2)

    @pl.when((bi == 0) & (c == 0))
    def _load_weights():
        cp = pltpu.make_async_copy(w_hbm, w_vmem, sem)
        cp.start()
        cp.wait()

    @pl.when(c == 0)
    def _reset_carry():
        carry_ref[...] = jnp.zeros_like(carry_ref)

    xb = (x_ref[0].astype(jnp.bfloat16)
          .reshape(R, G, D).swapaxes(0, 1).reshape(T, D))

    logits = jnp.dot(xb, wgate_ref[...],
                     preferred_element_type=jnp.float32)[:, :E]
    m1 = jnp.max(logits, axis=-1, keepdims=True)
    masked = jnp.where(logits >= m1, -1e30, logits)
    m2 = jnp.max(masked, axis=-1, keepdims=True)
    denom = 1.0 + jnp.exp(m2 - m1)
    wts = jnp.where(logits >= m2, jnp.exp(logits - m1), 0.0) / denom

    y = jnp.zeros((T, D), jnp.float32)
    for e in range(E):
        gvd = jnp.dot(xb, w_vmem[e], preferred_element_type=jnp.float32)
        g = gvd[:, 0:D] + bg_ref[e:e + 1, :]
        v = gvd[:, D:2 * D] + bv_ref[e:e + 1, :]
        d = gvd[:, 2 * D:3 * D] + bd_ref[e:e + 1, :]
        X = (0.5 * jnp.tanh(0.5 * g) + 0.5) * jnp.tanh(v)
        A = 0.5 + 0.499 * jnp.tanh(0.5 * d)

        kk = R
        while kk < T:
            X = jnp.concatenate([X[:kk], X[kk:] + A[kk:] * X[:T - kk]], axis=0)
            A = jnp.concatenate([A[:kk], A[kk:] * A[:T - kk]], axis=0)
            kk *= 2

        sA = A[T - R:]
        sX = X[T - R:]
        for k in (1, 2, 4):
            shA = jnp.concatenate(
                [jnp.ones((k, D), jnp.float32), sA[:R - k]], axis=0)
            shX = jnp.concatenate(
                [jnp.zeros((k, D), jnp.float32), sX[:R - k]], axis=0)
            sX = sX + sA * shX
            sA = sA * shA
        pA = jnp.concatenate([jnp.ones((1, D), jnp.float32), sA[:R - 1]], axis=0)
        pX = jnp.concatenate([jnp.zeros((1, D), jnp.float32), sX[:R - 1]], axis=0)
        hprev = carry_ref[e:e + 1, :]
        P = pX + pA * hprev

        out_e = (X.reshape(G, R, D) + A.reshape(G, R, D) * P[None, :, :]
                 ).reshape(T, D)
        carry_ref[e:e + 1, :] = out_e[T - 1:T, :]
        y = y + out_e * wts[:, e:e + 1]
    o_ref[0] = y.reshape(G, R, D).swapaxes(0, 1).reshape(T, D)


def kernel(x, Wg, bg, Wv, bv, Wd, bd, Wgate):
    B, S, d_model = x.shape
    assert d_model == D and B % 2 == 0 and S % T == 0
    n_chunks = S // T

    xp = x

    w_stack = jnp.concatenate(
        [jnp.swapaxes(Wg, 1, 2), jnp.swapaxes(Wv, 1, 2),
         jnp.swapaxes(Wd, 1, 2)], axis=2).astype(jnp.bfloat16)
    wgate_pad = jnp.zeros((D, 128), jnp.float32).at[:, :E].set(Wgate.T)
    wgate_pad = wgate_pad.astype(jnp.bfloat16)

    grid = (2, B // 2, n_chunks)

    out = pl.pallas_call(
        _body,
        out_shape=jax.ShapeDtypeStruct((B, S, D), jnp.float32),
        grid=grid,
        in_specs=[
            pl.BlockSpec((1, T, D), lambda bo, bi, c: (bo * 2 + bi, c, 0)),
            pl.BlockSpec(memory_space=pl.ANY),
            pl.BlockSpec((D, 128), lambda bo, bi, c: (0, 0)),
            pl.BlockSpec((E, D), lambda bo, bi, c: (0, 0)),
            pl.BlockSpec((E, D), lambda bo, bi, c: (0, 0)),
            pl.BlockSpec((E, D), lambda bo, bi, c: (0, 0)),
        ],
        out_specs=pl.BlockSpec((1, T, D), lambda bo, bi, c: (bo * 2 + bi, c, 0)),
        scratch_shapes=[
            pltpu.VMEM((E, D, 3 * D), jnp.bfloat16),
            pltpu.VMEM((E, D), jnp.float32),
            pltpu.SemaphoreType.DMA,
        ],
        compiler_params=pltpu.CompilerParams(
            dimension_semantics=("parallel", "arbitrary", "arbitrary"),
            vmem_limit_bytes=56 * 1024 * 1024,
        ),
        name="moe_mingru",
    )(xp, w_stack, wgate_pad, bg, bv, bd)

    return out


# speedup vs baseline: 64.7331x; 1.1662x over previous
import jax
import jax.numpy as jnp
import numpy as np
from jax.experimental import pallas as pl
from jax.experimental.pallas import tpu as pltpu

E = 4
D = 1024
T = 256
R = 8
G = T // R


def _body(x_ref, w_hbm, wgate_ref, bg_ref, bv_ref, bd_ref, o_ref,
          w_vmem, carry_ref, sem):
    bi = pl.program_id(1)
    c = pl.program_id(2)

    @pl.when((bi == 0) & (c == 0))
    def _load_weights():
        cp = pltpu.make_async_copy(w_hbm, w_vmem, sem)
        cp.start()
        cp.wait()

    @pl.when(c == 0)
    def _reset_carry():
        carry_ref[...] = jnp.zeros_like(carry_ref)

    xb = (x_ref[0].astype(jnp.bfloat16)
          .reshape(R, G, D).swapaxes(0, 1).reshape(T, D))

    logits = jnp.dot(xb, wgate_ref[...],
                     preferred_element_type=jnp.float32)[:, :E]
    m1 = jnp.max(logits, axis=-1, keepdims=True)
    masked = jnp.where(logits >= m1, -1e30, logits)
    m2 = jnp.max(masked, axis=-1, keepdims=True)
    denom = 1.0 + jnp.exp(m2 - m1)
    wts = jnp.where(logits >= m2, jnp.exp(logits - m1), 0.0) / denom

    y = jnp.zeros((T, D), jnp.float32)
    for e in range(E):
        gvd = jnp.dot(xb, w_vmem[e], preferred_element_type=jnp.float32)
        g = gvd[:, 0:D] + bg_ref[e:e + 1, :]
        v = gvd[:, D:2 * D] + bv_ref[e:e + 1, :]
        d = gvd[:, 2 * D:3 * D] + bd_ref[e:e + 1, :]
        X = (0.5 * jnp.tanh(0.5 * g) + 0.5) * jnp.tanh(v)
        A = 0.5 + 0.499 * jnp.tanh(0.5 * d)

        a_t = A
        hs = []
        ps = []
        h = X[0:R]
        p = a_t[0:R]
        hs.append(h)
        ps.append(p)
        for gi in range(1, G):
            sl = slice(R * gi, R * gi + R)
            h = a_t[sl] * h + X[sl]
            p = a_t[sl] * p
            hs.append(h)
            ps.append(p)
        X = jnp.concatenate(hs, axis=0)
        A = jnp.concatenate(ps, axis=0)

        sA = A[T - R:]
        sX = X[T - R:]
        for k in (1, 2, 4):
            shA = jnp.concatenate(
                [jnp.ones((k, D), jnp.float32), sA[:R - k]], axis=0)
            shX = jnp.concatenate(
                [jnp.zeros((k, D), jnp.float32), sX[:R - k]], axis=0)
            sX = sX + sA * shX
            sA = sA * shA
        pA = jnp.concatenate([jnp.ones((1, D), jnp.float32), sA[:R - 1]], axis=0)
        pX = jnp.concatenate([jnp.zeros((1, D), jnp.float32), sX[:R - 1]], axis=0)
        hprev = carry_ref[e:e + 1, :]
        P = pX + pA * hprev

        out_e = (X.reshape(G, R, D) + A.reshape(G, R, D) * P[None, :, :]
                 ).reshape(T, D)
        carry_ref[e:e + 1, :] = out_e[T - 1:T, :]
        y = y + out_e * wts[:, e:e + 1]
    o_ref[0] = y.reshape(G, R, D).swapaxes(0, 1).reshape(T, D)


def kernel(x, Wg, bg, Wv, bv, Wd, bd, Wgate):
    B, S, d_model = x.shape
    assert d_model == D and B % 2 == 0 and S % T == 0
    n_chunks = S // T

    xp = x

    w_stack = jnp.concatenate(
        [jnp.swapaxes(Wg, 1, 2), jnp.swapaxes(Wv, 1, 2),
         jnp.swapaxes(Wd, 1, 2)], axis=2).astype(jnp.bfloat16)
    wgate_pad = jnp.zeros((D, 128), jnp.float32).at[:, :E].set(Wgate.T)
    wgate_pad = wgate_pad.astype(jnp.bfloat16)

    grid = (2, B // 2, n_chunks)

    out = pl.pallas_call(
        _body,
        out_shape=jax.ShapeDtypeStruct((B, S, D), jnp.float32),
        grid=grid,
        in_specs=[
            pl.BlockSpec((1, T, D), lambda bo, bi, c: (bo * 2 + bi, c, 0)),
            pl.BlockSpec(memory_space=pl.ANY),
            pl.BlockSpec((D, 128), lambda bo, bi, c: (0, 0)),
            pl.BlockSpec((E, D), lambda bo, bi, c: (0, 0)),
            pl.BlockSpec((E, D), lambda bo, bi, c: (0, 0)),
            pl.BlockSpec((E, D), lambda bo, bi, c: (0, 0)),
        ],
        out_specs=pl.BlockSpec((1, T, D), lambda bo, bi, c: (bo * 2 + bi, c, 0)),
        scratch_shapes=[
            pltpu.VMEM((E, D, 3 * D), jnp.bfloat16),
            pltpu.VMEM((E, D), jnp.float32),
            pltpu.SemaphoreType.DMA,
        ],
        compiler_params=pltpu.CompilerParams(
            dimension_semantics=("parallel", "arbitrary", "arbitrary"),
            vmem_limit_bytes=56 * 1024 * 1024,
        ),
        name="moe_mingru",
    )(xp, w_stack, wgate_pad, bg, bv, bd)

    return out


# speedup vs baseline: 68.5396x; 1.0588x over previous
import jax
import jax.numpy as jnp
import numpy as np
from jax.experimental import pallas as pl
from jax.experimental.pallas import tpu as pltpu

E = 4
D = 1024
T = 256
R = 8
G = T // R


def _body(x_ref, w_hbm, wgate_ref, bg_ref, bv_ref, bd_ref, o_ref,
          w_vmem, carry_ref, sem):
    bi = pl.program_id(1)
    c = pl.program_id(2)

    @pl.when((bi == 0) & (c == 0))
    def _load_weights():
        cp = pltpu.make_async_copy(w_hbm, w_vmem, sem)
        cp.start()
        cp.wait()

    @pl.when(c == 0)
    def _reset_carry():
        carry_ref[...] = jnp.zeros_like(carry_ref)

    xb = (x_ref[0].astype(jnp.bfloat16)
          .reshape(R, G, D).swapaxes(0, 1).reshape(T, D))

    logits = jnp.dot(xb, wgate_ref[...],
                     preferred_element_type=jnp.float32)[:, :E]
    m1 = jnp.max(logits, axis=-1, keepdims=True)
    masked = jnp.where(logits >= m1, -1e30, logits)
    m2 = jnp.max(masked, axis=-1, keepdims=True)
    denom = 1.0 + jnp.exp(m2 - m1)
    wts = jnp.where(logits >= m2, jnp.exp(logits - m1), 0.0) / denom

    y = jnp.zeros((T, D), jnp.float32)
    for e in range(E):
        gvd = jnp.dot(xb, w_vmem[e], preferred_element_type=jnp.float32)
        g = gvd[:, 0:D] + bg_ref[e:e + 1, :]
        v = gvd[:, D:2 * D] + bv_ref[e:e + 1, :]
        d = gvd[:, 2 * D:3 * D] + bd_ref[e:e + 1, :]
        X = (0.5 * jnp.tanh(g) + 0.5) * jnp.tanh(v)
        A = 0.5 + 0.499 * jnp.tanh(d)

        a_t = A
        h = X[0:R]
        p = a_t[0:R]
        for gi in range(1, G):
            sl = slice(R * gi, R * gi + R)
            h = a_t[sl] * h + X[sl]
            p = a_t[sl] * p
        sX, sA = h, p

        for k in (1, 2, 4):
            shA = jnp.concatenate(
                [jnp.ones((k, D), jnp.float32), sA[:R - k]], axis=0)
            shX = jnp.concatenate(
                [jnp.zeros((k, D), jnp.float32), sX[:R - k]], axis=0)
            sX = sX + sA * shX
            sA = sA * shA
        pA = jnp.concatenate([jnp.ones((1, D), jnp.float32), sA[:R - 1]], axis=0)
        pX = jnp.concatenate([jnp.zeros((1, D), jnp.float32), sX[:R - 1]], axis=0)
        hprev = carry_ref[e:e + 1, :]
        P = pX + pA * hprev

        w_e = wts[:, e:e + 1]
        h = P
        ys = []
        for gi in range(G):
            sl = slice(R * gi, R * gi + R)
            h = a_t[sl] * h + X[sl]
            ys.append(h * w_e[sl])
        carry_ref[e:e + 1, :] = h[R - 1:R, :]
        y = y + jnp.concatenate(ys, axis=0)
    o_ref[0] = y.reshape(G, R, D).swapaxes(0, 1).reshape(T, D)


def kernel(x, Wg, bg, Wv, bv, Wd, bd, Wgate):
    B, S, d_model = x.shape
    assert d_model == D and B % 2 == 0 and S % T == 0
    n_chunks = S // T

    xp = x

    w_stack = jnp.concatenate(
        [jnp.swapaxes(Wg, 1, 2) * 0.5, jnp.swapaxes(Wv, 1, 2),
         jnp.swapaxes(Wd, 1, 2) * 0.5], axis=2).astype(jnp.bfloat16)
    wgate_pad = jnp.zeros((D, 128), jnp.float32).at[:, :E].set(Wgate.T)
    wgate_pad = wgate_pad.astype(jnp.bfloat16)

    grid = (2, B // 2, n_chunks)

    out = pl.pallas_call(
        _body,
        out_shape=jax.ShapeDtypeStruct((B, S, D), jnp.float32),
        grid=grid,
        in_specs=[
            pl.BlockSpec((1, T, D), lambda bo, bi, c: (bo * 2 + bi, c, 0)),
            pl.BlockSpec(memory_space=pl.ANY),
            pl.BlockSpec((D, 128), lambda bo, bi, c: (0, 0)),
            pl.BlockSpec((E, D), lambda bo, bi, c: (0, 0)),
            pl.BlockSpec((E, D), lambda bo, bi, c: (0, 0)),
            pl.BlockSpec((E, D), lambda bo, bi, c: (0, 0)),
        ],
        out_specs=pl.BlockSpec((1, T, D), lambda bo, bi, c: (bo * 2 + bi, c, 0)),
        scratch_shapes=[
            pltpu.VMEM((E, D, 3 * D), jnp.bfloat16),
            pltpu.VMEM((E, D), jnp.float32),
            pltpu.SemaphoreType.DMA,
        ],
        compiler_params=pltpu.CompilerParams(
            dimension_semantics=("parallel", "arbitrary", "arbitrary"),
            vmem_limit_bytes=56 * 1024 * 1024,
        ),
        name="moe_mingru",
    )(xp, w_stack, wgate_pad, bg * 0.5, bv, bd * 0.5)

    return out


# speedup vs baseline: 68.5580x; 1.0003x over previous
import jax
import jax.numpy as jnp
from jax.experimental import pallas as pl
from jax.experimental.pallas import tpu as pltpu

E = 4
D = 1024
T = 256
R = 8
G = T // R


def _body(x_ref, w_hbm, wgate_ref, bg_ref, bv_ref, bd_ref, o_ref,
          w_vmem, carry_ref, sem):
    bi = pl.program_id(1)
    c = pl.program_id(2)

    @pl.when((bi == 0) & (c == 0))
    def _load_weights():
        cp = pltpu.make_async_copy(w_hbm, w_vmem, sem)
        cp.start()
        cp.wait()

    @pl.when(c == 0)
    def _reset_carry():
        carry_ref[...] = jnp.zeros_like(carry_ref)

    xb = (x_ref[0].astype(jnp.bfloat16)
          .reshape(R, G, D).swapaxes(0, 1).reshape(T, D))

    logits = jnp.dot(xb, wgate_ref[...],
                     preferred_element_type=jnp.float32)[:, :E]
    m1 = jnp.max(logits, axis=-1, keepdims=True)
    masked = jnp.where(logits >= m1, -1e30, logits)
    m2 = jnp.max(masked, axis=-1, keepdims=True)
    denom = 1.0 + jnp.exp(m2 - m1)
    wts = jnp.where(logits >= m2, jnp.exp(logits - m1), 0.0) / denom

    y = jnp.zeros((T, D), jnp.float32)
    for e in range(E):
        gvd = jnp.dot(xb, w_vmem[e], preferred_element_type=jnp.float32)
        g = gvd[:, 0:D] + bg_ref[e:e + 1, :]
        v = gvd[:, D:2 * D] + bv_ref[e:e + 1, :]
        d = gvd[:, 2 * D:3 * D] + bd_ref[e:e + 1, :]
        X = (0.5 * jnp.tanh(g) + 0.5) * jnp.tanh(v)
        A = 0.5 + 0.499 * jnp.tanh(d)

        a_t = A
        h = X[0:R]
        p = a_t[0:R]
        for gi in range(1, G):
            sl = slice(R * gi, R * gi + R)
            h = a_t[sl] * h + X[sl]
            p = a_t[sl] * p
        sX, sA = h, p

        for k in (1, 2, 4):
            shA = jnp.concatenate(
                [jnp.ones((k, D), jnp.float32), sA[:R - k]], axis=0)
            shX = jnp.concatenate(
                [jnp.zeros((k, D), jnp.float32), sX[:R - k]], axis=0)
            sX = sX + sA * shX
            sA = sA * shA
        pA = jnp.concatenate([jnp.ones((1, D), jnp.float32), sA[:R - 1]], axis=0)
        pX = jnp.concatenate([jnp.zeros((1, D), jnp.float32), sX[:R - 1]], axis=0)
        hprev = carry_ref[e:e + 1, :]
        P = pX + pA * hprev

        w_e = wts[:, e:e + 1]
        h = P
        ys = []
        for gi in range(G):
            sl = slice(R * gi, R * gi + R)
            h = a_t[sl] * h + X[sl]
            ys.append(h * w_e[sl])
        carry_ref[e:e + 1, :] = h[R - 1:R, :]
        y = y + jnp.concatenate(ys, axis=0)
    o_ref[0] = y.reshape(G, R, D).swapaxes(0, 1).reshape(T, D)


def kernel(x, Wg, bg, Wv, bv, Wd, bd, Wgate):
    B, S, d_model = x.shape
    assert d_model == D and B % 2 == 0 and S % T == 0
    n_chunks = S // T

    xp = x

    w_stack = jnp.concatenate(
        [jnp.swapaxes(Wg, 1, 2) * 0.5, jnp.swapaxes(Wv, 1, 2),
         jnp.swapaxes(Wd, 1, 2) * 0.5], axis=2).astype(jnp.bfloat16)
    wgate_pad = jnp.zeros((D, 128), jnp.float32).at[:, :E].set(Wgate.T)
    wgate_pad = wgate_pad.astype(jnp.bfloat16)

    grid = (2, B // 2, n_chunks)

    out = pl.pallas_call(
        _body,
        out_shape=jax.ShapeDtypeStruct((B, S, D), jnp.float32),
        grid=grid,
        in_specs=[
            pl.BlockSpec((1, T, D), lambda bo, bi, c: (bo * 2 + bi, c, 0)),
            pl.BlockSpec(memory_space=pl.ANY),
            pl.BlockSpec((D, 128), lambda bo, bi, c: (0, 0)),
            pl.BlockSpec((E, D), lambda bo, bi, c: (0, 0)),
            pl.BlockSpec((E, D), lambda bo, bi, c: (0, 0)),
            pl.BlockSpec((E, D), lambda bo, bi, c: (0, 0)),
        ],
        out_specs=pl.BlockSpec((1, T, D), lambda bo, bi, c: (bo * 2 + bi, c, 0)),
        scratch_shapes=[
            pltpu.VMEM((E, D, 3 * D), jnp.bfloat16),
            pltpu.VMEM((E, D), jnp.float32),
            pltpu.SemaphoreType.DMA,
        ],
        compiler_params=pltpu.CompilerParams(
            dimension_semantics=("parallel", "arbitrary", "arbitrary"),
            vmem_limit_bytes=56 * 1024 * 1024,
        ),
        name="moe_mingru",
    )(xp, w_stack, wgate_pad, bg * 0.5, bv, bd * 0.5)

    return out
